# baseline (speedup 1.0000x reference)
import jax
import jax.numpy as jnp
from jax.experimental import pallas as pl
from jax.experimental.pallas import tpu as pltpu

ANCHOR_SIZES = (8, 16, 32, 64)
BATCH = 16
TIME = 512
FEAT = 128
SEG = 64
REP = 512
NCLS = 20
TOPK = 64
BT = 4
ROWS = 4 * BT * SEG


def _main_kernel(starts_ref, f0, f1, f2, f3, cw_ref, cb_ref, l1_ref, b1_ref,
                 l2_ref, b2_ref, s_ref, e_ref, cas_ref,
                 scr0, scr1, scr2, scr3, sem):
    pid = pl.program_id(0)
    dot = lambda a, b: jax.lax.dot_general(
        a, b, (((1,), (0,)), ((), ())), preferred_element_type=jnp.float32)

    copies = []
    for i, L, f_ref, scr in ((2, 32, f2, scr2), (3, 64, f3, scr3)):
        for bb in range(BT):
            for s in range(SEG):
                st = starts_ref[i, pid * BT + bb, s]
                cp = pltpu.make_async_copy(
                    f_ref.at[bb, pl.ds(st, L), :],
                    scr.at[:, bb * SEG + s, :], sem)
                cp.start()
                copies.append(cp)
    for i, (L, f_ref, scr) in enumerate(
            zip(ANCHOR_SIZES[:2], (f0, f1), (scr0, scr1))):
        for bb in range(BT):
            for s in range(SEG):
                st = starts_ref[i, pid * BT + bb, s]
                scr[:, bb * SEG + s, :] = f_ref[bb, pl.ds(st, L), :]

    nseg = BT * SEG
    cb = cb_ref[0]

    fi = jax.lax.broadcasted_iota(jnp.int32, (FEAT, SEG), 0)
    ui = jax.lax.broadcasted_iota(jnp.int32, (FEAT, SEG), 1)
    e2 = (fi == 2 * ui).astype(jnp.float32)
    fi3 = jax.lax.broadcasted_iota(jnp.int32, (FEAT, 48), 0)
    qi3 = jax.lax.broadcasted_iota(jnp.int32, (FEAT, 48), 1)
    r3 = [(fi3 == 3 * qi3 + r).astype(jnp.float32) for r in range(3)]

    x0 = jnp.zeros((nseg, REP), jnp.float32)
    x1 = jnp.zeros((nseg, REP), jnp.float32)
    for t in range(16):
        w = cw_ref[128 * t:128 * (t + 1), :]
        if t < 8:
            x01 = dot(jnp.concatenate([scr0[t], scr1[t]], axis=0), w)
            x0 += x01[:nseg]
            x1 += x01[nseg:]
        else:
            x1 += dot(scr1[t], w)

    for cp in copies:
        cp.wait()

    x2 = jnp.zeros((nseg, REP), jnp.float32)
    for t in range(32):
        row = scr2[t]
        s1 = jnp.concatenate([row[:, 1:], row[:, 0:1]], axis=1)
        p2 = dot(jnp.maximum(row, s1), e2)
        x2 += dot(p2, cw_ref[64 * t:64 * (t + 1), :])

    x3 = jnp.zeros((nseg, REP), jnp.float32)
    for t in range(64):
        row = scr3[t]
        if t < 63:
            nxt = scr3[t + 1][:, 0:2]
        else:
            nxt = jnp.zeros((nseg, 2), jnp.float32)
        s1 = jnp.concatenate([row[:, 1:], nxt[:, 0:1]], axis=1)
        s2 = jnp.concatenate([row[:, 2:], nxt], axis=1)
        m3 = jnp.maximum(jnp.maximum(row, s1), s2)
        p3 = dot(m3, r3[t % 3])
        base = (128 * t + t % 3) // 3
        x3 += dot(p3, cw_ref[base:base + 48, :])

    xs = jnp.concatenate([x0, x1, x2, x3], axis=0) + cb
    xs = jax.nn.relu(xs)
    h = dot(xs, l1_ref[...]) + b1_ref[0]
    sc = jax.nn.relu(dot(h, l2_ref[...]) + b2_ref[0])
    sc1 = jnp.concatenate([sc, jnp.ones((ROWS, 1), jnp.float32)], axis=1)

    ti = jax.lax.broadcasted_iota(jnp.int32, (ROWS, TIME), 1)
    m_full = (ti >= s_ref[0]) & (ti < e_ref[0])
    rbb = jax.lax.broadcasted_iota(jnp.int32, (ROWS, 1), 0) // SEG % BT
    for bb in range(BT):
        m_bb = (m_full & (rbb == bb)).astype(jnp.float32)
        ce = jax.lax.dot_general(m_bb, sc1, (((0,), (0,)), ((), ())),
                                 preferred_element_type=jnp.float32)
        cnt = ce[:, NCLS:]
        cnt = jnp.where(cnt == 0.0, 1.0, cnt)
        cas_ref[bb] = ce[:, :NCLS] / cnt


def _topk_kernel(casT_ref, kf_ref, out_ref):
    casT = casT_ref[...]
    bits = jax.lax.bitcast_convert_type(casT, jnp.int32)
    ones = jnp.ones((1, TIME), jnp.float32)
    cdot = lambda a: jax.lax.dot_general(
        ones, a, (((1,), (0,)), ((), ())),
        preferred_element_type=jnp.float32)
    th = jnp.zeros((1, BATCH * NCLS), jnp.int32)
    for bit in range(30, -1, -1):
        cand = th | (1 << bit)
        n_ge = cdot((bits >= cand).astype(jnp.float32))
        th = jnp.where(n_ge >= float(TOPK), cand, th)
    thf = jax.lax.bitcast_convert_type(th, jnp.float32)
    gt = (casT > thf).astype(jnp.float32)
    s_gt = cdot(casT * gt)
    n_gt = cdot(gt)
    ts = s_gt + thf * (float(TOPK) - n_gt)

    v = ts / kf_ref[0, 0]
    v = v - jnp.max(v)
    e = jnp.exp(v)
    gi = jax.lax.broadcasted_iota(jnp.int32, (BATCH * NCLS,) * 2, 0) // NCLS
    gj = jax.lax.broadcasted_iota(jnp.int32, (BATCH * NCLS,) * 2, 1) // NCLS
    gg = (gi == gj).astype(jnp.float32)
    gs = jax.lax.dot_general(e, gg, (((1,), (0,)), ((), ())),
                             preferred_element_type=jnp.float32)
    out_ref[...] = e / gs


def kernel(features, proposals, conv_w, conv_b, lin1_w, lin1_b, lin2_w,
           lin2_b, k):
    starts = proposals[..., 0]
    ends = proposals[..., 1]

    s_arr = (starts.reshape(4, BATCH // BT, BT, SEG).transpose(1, 0, 2, 3)
             .reshape(BATCH // BT, ROWS, 1))
    e_arr = (ends.reshape(4, BATCH // BT, BT, SEG).transpose(1, 0, 2, 3)
             .reshape(BATCH // BT, ROWS, 1))

    cas = pl.pallas_call(
        _main_kernel,
        grid=(BATCH // BT,),
        in_specs=[
            pl.BlockSpec(memory_space=pltpu.SMEM),
            pl.BlockSpec((BT, TIME, FEAT), lambda p: (p, 0, 0)),
            pl.BlockSpec((BT, TIME, FEAT), lambda p: (p, 0, 0)),
            pl.BlockSpec((BT, TIME, FEAT), lambda p: (p, 0, 0)),
            pl.BlockSpec((BT, TIME, FEAT), lambda p: (p, 0, 0)),
            pl.BlockSpec((4096, REP), lambda p: (0, 0)),
            pl.BlockSpec((1, REP), lambda p: (0, 0)),
            pl.BlockSpec((REP, REP), lambda p: (0, 0)),
            pl.BlockSpec((1, REP), lambda p: (0, 0)),
            pl.BlockSpec((REP, NCLS), lambda p: (0, 0)),
            pl.BlockSpec((1, NCLS), lambda p: (0, 0)),
            pl.BlockSpec((1, ROWS, 1), lambda p: (p, 0, 0)),
            pl.BlockSpec((1, ROWS, 1), lambda p: (p, 0, 0)),
        ],
        out_specs=pl.BlockSpec((BT, TIME, NCLS), lambda p: (p, 0, 0)),
        out_shape=jax.ShapeDtypeStruct((BATCH, TIME, NCLS), jnp.float32),
        scratch_shapes=[
            pltpu.VMEM((L, BT * SEG, FEAT), jnp.float32) for L in ANCHOR_SIZES
        ] + [pltpu.SemaphoreType.DMA],
    )(starts, features[0], features[1], features[2], features[3],
      conv_w, conv_b.reshape(1, REP), lin1_w, lin1_b.reshape(1, REP),
      lin2_w, lin2_b.reshape(1, NCLS), s_arr, e_arr)

    casT = cas.transpose(1, 0, 2).reshape(TIME, BATCH * NCLS)
    kf = jnp.asarray(k, jnp.float32).reshape(1, 1)
    out = pl.pallas_call(
        _topk_kernel,
        grid=(1,),
        in_specs=[pl.BlockSpec((TIME, BATCH * NCLS), lambda i: (0, 0)),
                  pl.BlockSpec(memory_space=pltpu.SMEM)],
        out_specs=pl.BlockSpec((1, BATCH * NCLS), lambda i: (0, 0)),
        out_shape=jax.ShapeDtypeStruct((1, BATCH * NCLS), jnp.float32),
    )(casT, kf)
    return out.reshape(BATCH, NCLS)


# speedup vs baseline: 1.1883x; 1.1883x over previous
import jax
import jax.numpy as jnp
from jax.experimental import pallas as pl
from jax.experimental.pallas import tpu as pltpu

ANCHOR_SIZES = (8, 16, 32, 64)
BATCH = 16
TIME = 512
FEAT = 128
SEG = 64
REP = 512
NCLS = 20
TOPK = 64
BT = 4
ROWS = 4 * BT * SEG


def _main_kernel(starts_ref, f0, f1, f2, f3, cw_ref, cb_ref, l1_ref, b1_ref,
                 l2_ref, b2_ref, s_ref, e_ref, cas_ref,
                 scr0, scr1, scr2, scr3):
    pid = pl.program_id(0)
    dot = lambda a, b: jax.lax.dot_general(
        a, b, (((1,), (0,)), ((), ())), preferred_element_type=jnp.float32)

    for i, (L, f_ref, scr) in enumerate(
            zip(ANCHOR_SIZES, (f0, f1, f2, f3), (scr0, scr1, scr2, scr3))):
        for bb in range(BT):
            for s in range(SEG):
                st = starts_ref[i, pid * BT + bb, s]
                scr[:, bb * SEG + s, :] = f_ref[bb, pl.ds(st, L), :]

    nseg = BT * SEG
    cb = cb_ref[0]

    fi = jax.lax.broadcasted_iota(jnp.int32, (FEAT, SEG), 0)
    ui = jax.lax.broadcasted_iota(jnp.int32, (FEAT, SEG), 1)
    e2 = (fi == 2 * ui).astype(jnp.float32)
    fi3 = jax.lax.broadcasted_iota(jnp.int32, (FEAT, 48), 0)
    qi3 = jax.lax.broadcasted_iota(jnp.int32, (FEAT, 48), 1)
    r3 = [(fi3 == 3 * qi3 + r).astype(jnp.float32) for r in range(3)]

    x0 = jnp.zeros((nseg, REP), jnp.float32)
    x1 = jnp.zeros((nseg, REP), jnp.float32)
    for t in range(16):
        w = cw_ref[128 * t:128 * (t + 1), :]
        if t < 8:
            x01 = dot(jnp.concatenate([scr0[t], scr1[t]], axis=0), w)
            x0 += x01[:nseg]
            x1 += x01[nseg:]
        else:
            x1 += dot(scr1[t], w)

    x2 = jnp.zeros((nseg, REP), jnp.float32)
    for t in range(32):
        row = scr2[t]
        s1 = jnp.concatenate([row[:, 1:], row[:, 0:1]], axis=1)
        p2 = dot(jnp.maximum(row, s1), e2)
        x2 += dot(p2, cw_ref[64 * t:64 * (t + 1), :])

    x3 = jnp.zeros((nseg, REP), jnp.float32)
    for t in range(64):
        row = scr3[t]
        if t < 63:
            nxt = scr3[t + 1][:, 0:2]
        else:
            nxt = jnp.zeros((nseg, 2), jnp.float32)
        s1 = jnp.concatenate([row[:, 1:], nxt[:, 0:1]], axis=1)
        s2 = jnp.concatenate([row[:, 2:], nxt], axis=1)
        m3 = jnp.maximum(jnp.maximum(row, s1), s2)
        p3 = dot(m3, r3[t % 3])
        base = (128 * t + t % 3) // 3
        x3 += dot(p3, cw_ref[base:base + 48, :])

    xs = jnp.concatenate([x0, x1, x2, x3], axis=0) + cb
    xs = jax.nn.relu(xs)
    h = dot(xs, l1_ref[...]) + b1_ref[0]
    sc = jax.nn.relu(dot(h, l2_ref[...]) + b2_ref[0])
    sc1 = jnp.concatenate([sc, jnp.ones((ROWS, 1), jnp.float32)], axis=1)

    ti = jax.lax.broadcasted_iota(jnp.int32, (ROWS, TIME), 1)
    m_full = (ti >= s_ref[0]) & (ti < e_ref[0])
    rbb = jax.lax.broadcasted_iota(jnp.int32, (ROWS, 1), 0) // SEG % BT
    for bb in range(BT):
        m_bb = (m_full & (rbb == bb)).astype(jnp.float32)
        ce = jax.lax.dot_general(m_bb, sc1, (((0,), (0,)), ((), ())),
                                 preferred_element_type=jnp.float32)
        cnt = ce[:, NCLS:]
        cnt = jnp.where(cnt == 0.0, 1.0, cnt)
        cas_ref[bb] = ce[:, :NCLS] / cnt


def _topk_kernel(casT_ref, kf_ref, out_ref):
    casT = casT_ref[...]
    bits = jax.lax.bitcast_convert_type(casT, jnp.int32)
    ones = jnp.ones((1, TIME), jnp.float32)
    cdot = lambda a: jax.lax.dot_general(
        ones, a, (((1,), (0,)), ((), ())),
        preferred_element_type=jnp.float32)
    th = jnp.zeros((1, BATCH * NCLS), jnp.int32)
    for bit in range(30, -1, -1):
        cand = th | (1 << bit)
        n_ge = cdot((bits >= cand).astype(jnp.float32))
        th = jnp.where(n_ge >= float(TOPK), cand, th)
    thf = jax.lax.bitcast_convert_type(th, jnp.float32)
    gt = (casT > thf).astype(jnp.float32)
    s_gt = cdot(casT * gt)
    n_gt = cdot(gt)
    ts = s_gt + thf * (float(TOPK) - n_gt)

    v = ts / kf_ref[0, 0]
    v = v - jnp.max(v)
    e = jnp.exp(v)
    gi = jax.lax.broadcasted_iota(jnp.int32, (BATCH * NCLS,) * 2, 0) // NCLS
    gj = jax.lax.broadcasted_iota(jnp.int32, (BATCH * NCLS,) * 2, 1) // NCLS
    gg = (gi == gj).astype(jnp.float32)
    gs = jax.lax.dot_general(e, gg, (((1,), (0,)), ((), ())),
                             preferred_element_type=jnp.float32)
    out_ref[...] = e / gs


def kernel(features, proposals, conv_w, conv_b, lin1_w, lin1_b, lin2_w,
           lin2_b, k):
    starts = proposals[..., 0]
    ends = proposals[..., 1]

    s_arr = (starts.reshape(4, BATCH // BT, BT, SEG).transpose(1, 0, 2, 3)
             .reshape(BATCH // BT, ROWS, 1))
    e_arr = (ends.reshape(4, BATCH // BT, BT, SEG).transpose(1, 0, 2, 3)
             .reshape(BATCH // BT, ROWS, 1))

    cas = pl.pallas_call(
        _main_kernel,
        grid=(BATCH // BT,),
        in_specs=[
            pl.BlockSpec(memory_space=pltpu.SMEM),
            pl.BlockSpec((BT, TIME, FEAT), lambda p: (p, 0, 0)),
            pl.BlockSpec((BT, TIME, FEAT), lambda p: (p, 0, 0)),
            pl.BlockSpec((BT, TIME, FEAT), lambda p: (p, 0, 0)),
            pl.BlockSpec((BT, TIME, FEAT), lambda p: (p, 0, 0)),
            pl.BlockSpec((4096, REP), lambda p: (0, 0)),
            pl.BlockSpec((1, REP), lambda p: (0, 0)),
            pl.BlockSpec((REP, REP), lambda p: (0, 0)),
            pl.BlockSpec((1, REP), lambda p: (0, 0)),
            pl.BlockSpec((REP, NCLS), lambda p: (0, 0)),
            pl.BlockSpec((1, NCLS), lambda p: (0, 0)),
            pl.BlockSpec((1, ROWS, 1), lambda p: (p, 0, 0)),
            pl.BlockSpec((1, ROWS, 1), lambda p: (p, 0, 0)),
        ],
        out_specs=pl.BlockSpec((BT, TIME, NCLS), lambda p: (p, 0, 0)),
        out_shape=jax.ShapeDtypeStruct((BATCH, TIME, NCLS), jnp.float32),
        scratch_shapes=[
            pltpu.VMEM((L, BT * SEG, FEAT), jnp.float32) for L in ANCHOR_SIZES
        ],
    )(starts, features[0], features[1], features[2], features[3],
      conv_w, conv_b.reshape(1, REP), lin1_w, lin1_b.reshape(1, REP),
      lin2_w, lin2_b.reshape(1, NCLS), s_arr, e_arr)

    casT = cas.transpose(1, 0, 2).reshape(TIME, BATCH * NCLS)
    kf = jnp.asarray(k, jnp.float32).reshape(1, 1)
    out = pl.pallas_call(
        _topk_kernel,
        grid=(1,),
        in_specs=[pl.BlockSpec((TIME, BATCH * NCLS), lambda i: (0, 0)),
                  pl.BlockSpec(memory_space=pltpu.SMEM)],
        out_specs=pl.BlockSpec((1, BATCH * NCLS), lambda i: (0, 0)),
        out_shape=jax.ShapeDtypeStruct((1, BATCH * NCLS), jnp.float32),
    )(casT, kf)
    return out.reshape(BATCH, NCLS)
